# baseline (device time: 135943 ns/iter reference)
import jax
import jax.numpy as jnp
from jax import lax
from jax.experimental import pallas as pl
from jax.experimental.pallas import tpu as pltpu

N_DEV = 4
C = 8


def kernel(x, pi):
    _, m, n = x.shape
    rows = m // C

    def body(x_ref, pi_ref, out_ref, load_buf, q_buf, s_buf, q_rcv, s_rcv,
             load_sems, dsend_sems, drecv_sems, ssend_sems, srecv_sems,
             credit_sem):
        my_pos = lax.axis_index("i")
        target = pi_ref[my_pos]
        sender = jnp.int32(0)
        for j in range(N_DEV):
            sender = jnp.where(pi_ref[j] == my_pos, jnp.int32(j), sender)

        def start_load(c):
            cp = pltpu.make_async_copy(
                x_ref.at[0, pl.ds(c * rows, rows), :],
                load_buf.at[c % 2],
                load_sems.at[c % 2],
            )
            cp.start()
            return cp

        def make_rdmas(c):
            slot = c % 2
            data = pltpu.make_async_remote_copy(
                src_ref=q_buf.at[slot],
                dst_ref=q_rcv.at[slot],
                send_sem=dsend_sems.at[slot],
                recv_sem=drecv_sems.at[slot],
                device_id=(target,),
                device_id_type=pl.DeviceIdType.MESH,
            )
            scales = pltpu.make_async_remote_copy(
                src_ref=s_buf.at[slot],
                dst_ref=s_rcv.at[slot],
                send_sem=ssend_sems.at[slot],
                recv_sem=srecv_sems.at[slot],
                device_id=(target,),
                device_id_type=pl.DeviceIdType.MESH,
            )
            return data, scales

        def consume(c, grant_credit):
            slot = c % 2
            rdmas[c][0].wait_recv()
            rdmas[c][1].wait_recv()
            out_ref[0, pl.ds(c * rows, rows), :] = (
                q_rcv[slot].astype(jnp.float32) * s_rcv[slot]
            )
            if grant_credit:
                pl.semaphore_signal(
                    credit_sem, inc=1,
                    device_id=(sender,),
                    device_id_type=pl.DeviceIdType.MESH,
                )

        loads = {0: start_load(0), 1: start_load(1)}
        rdmas = []
        for c in range(C):
            slot = c % 2
            loads[c].wait()
            if c >= 2:
                rdmas[c - 2][0].wait_send()
                rdmas[c - 2][1].wait_send()
            a = load_buf[slot]
            am = jnp.maximum(jnp.max(jnp.abs(a), axis=0, keepdims=True), 1e-30)
            rs = 127.0 / am
            q_buf[slot] = jnp.round(a * rs).astype(jnp.int8)
            s_buf[slot] = am * (1.0 / 127.0)
            if c + 2 < C:
                loads[c + 2] = start_load(c + 2)
            if c >= 2:
                pl.semaphore_wait(credit_sem, 1)
            rdmas.append(make_rdmas(c))
            rdmas[c][0].start()
            rdmas[c][1].start()
            if c >= 1:
                consume(c - 1, grant_credit=(c - 1) + 2 < C)
        consume(C - 1, grant_credit=False)
        for c in (C - 2, C - 1):
            rdmas[c][0].wait_send()
            rdmas[c][1].wait_send()

    return pl.pallas_call(
        body,
        out_shape=jax.ShapeDtypeStruct(x.shape, jnp.float32),
        in_specs=[
            pl.BlockSpec(memory_space=pl.ANY),
            pl.BlockSpec(memory_space=pltpu.SMEM),
        ],
        out_specs=pl.BlockSpec(memory_space=pltpu.VMEM),
        scratch_shapes=[
            pltpu.VMEM((2, rows, n), x.dtype),
            pltpu.VMEM((2, rows, n), jnp.int8),
            pltpu.VMEM((2, 1, n), jnp.float32),
            pltpu.VMEM((2, rows, n), jnp.int8),
            pltpu.VMEM((2, 1, n), jnp.float32),
            pltpu.SemaphoreType.DMA((2,)),
            pltpu.SemaphoreType.DMA((2,)),
            pltpu.SemaphoreType.DMA((2,)),
            pltpu.SemaphoreType.DMA((2,)),
            pltpu.SemaphoreType.DMA((2,)),
            pltpu.SemaphoreType.REGULAR,
        ],
        compiler_params=pltpu.CompilerParams(
            vmem_limit_bytes=100 * 1024 * 1024,
        ),
    )(x, pi)


# device time: 106945 ns/iter; 1.2711x vs baseline; 1.2711x over previous
import jax
import jax.numpy as jnp
from jax import lax
from jax.experimental import pallas as pl
from jax.experimental.pallas import tpu as pltpu

N_DEV = 4
U = 4

RIDX = {(0, 0): 0, (0, 1): 1, (0, 2): 2, (1, 1): 3, (1, 2): 4, (2, 2): 5}


def kernel(x, pi):
    _, m, n = x.shape
    urows = m // U
    prow = urows + 4

    def body(x_ref, pi_ref, out_ref, load_buf, ubuf, lrcv, rbuf,
             load_sems, lsend_sems, lrecv_sems, fsend_sems, frecv_sems):
        my = lax.axis_index("i")
        target = pi_ref[my]
        r = lax.rem(my - target + N_DEV, N_DEV)
        left_n = N_DEV - r
        rn = lax.rem(my + 1, N_DEV)

        def start_load(q, slot):
            cp = pltpu.make_async_copy(
                x_ref.at[0, pl.ds(q * urows, urows), :],
                load_buf.at[slot],
                load_sems.at[slot],
            )
            cp.start()
            return cp

        def dequant(buf, q):
            data = buf[pl.ds(0, urows), :]
            sc = pltpu.bitcast(buf[pl.ds(urows, 4), :], jnp.float32)
            out_ref[0, pl.ds(q * urows, urows), :] = (
                data.astype(jnp.bfloat16) * sc.astype(jnp.bfloat16)
            )

        order = (0, 3, 1, 2)
        loads = {0: start_load(order[0], 0), 1: start_load(order[1], 1)}
        left_rdma = {}
        own_rdma = {}
        for i in range(U):
            q = order[i]
            slot = i % 2
            loads[i].wait()
            a = load_buf[slot]
            am = jnp.maximum(jnp.max(jnp.abs(a), axis=0, keepdims=True), 1e-30)
            ubuf[q, pl.ds(0, urows), :] = (
                jnp.round(a * (127.0 / am)).astype(jnp.int8)
            )
            ubuf[q, pl.ds(urows, 4), :] = pltpu.bitcast(
                am * (1.0 / 127.0), jnp.int8
            )
            if i + 2 < U:
                loads[i + 2] = start_load(order[i + 2], slot)
            if q <= 2:
                dl = pltpu.make_async_remote_copy(
                    src_ref=ubuf.at[q],
                    dst_ref=lrcv.at[q],
                    send_sem=lsend_sems.at[q],
                    recv_sem=lrecv_sems.at[q],
                    device_id=(target,),
                    device_id_type=pl.DeviceIdType.MESH,
                )
                left_rdma[q] = dl

                @pl.when(q < left_n)
                def _():
                    dl.start()

            if q >= 1:
                do = pltpu.make_async_remote_copy(
                    src_ref=ubuf.at[q],
                    dst_ref=rbuf.at[RIDX[(0, q - 1)]],
                    send_sem=fsend_sems.at[0, q - 1],
                    recv_sem=frecv_sems.at[0, q - 1],
                    device_id=(rn,),
                    device_id_type=pl.DeviceIdType.MESH,
                )
                own_rdma[q] = do

                @pl.when(q >= left_n)
                def _():
                    do.start()

        fwd_rdma = {}
        for p in range(3):
            ql = p
            dl = left_rdma[ql]

            @pl.when(ql < left_n)
            def _():
                dl.wait_recv()
                dequant(lrcv.at[ql], ql)

            d = p + 1
            for q in (1, 2, 3):
                if d > q:
                    continue
                lives = (q >= left_n) & (d <= left_n)
                arr = pltpu.make_async_remote_copy(
                    src_ref=rbuf.at[RIDX[(d - 1, q - 1)]],
                    dst_ref=rbuf.at[RIDX[(d - 1, q - 1)]],
                    send_sem=fsend_sems.at[d - 1, q - 1],
                    recv_sem=frecv_sems.at[d - 1, q - 1],
                    device_id=(rn,),
                    device_id_type=pl.DeviceIdType.MESH,
                )

                @pl.when(lives)
                def _():
                    arr.wait_recv()

                if d <= 2 and d <= q - 1:
                    fw = pltpu.make_async_remote_copy(
                        src_ref=rbuf.at[RIDX[(d - 1, q - 1)]],
                        dst_ref=rbuf.at[RIDX[(d, q - 1)]],
                        send_sem=fsend_sems.at[d, q - 1],
                        recv_sem=frecv_sems.at[d, q - 1],
                        device_id=(rn,),
                        device_id_type=pl.DeviceIdType.MESH,
                    )
                    fwd_rdma[(d, q)] = fw

                    @pl.when((q >= left_n) & (d < left_n))
                    def _():
                        fw.start()

                @pl.when((q >= left_n) & (d == left_n))
                def _():
                    dequant(rbuf.at[RIDX[(d - 1, q - 1)]], q)

        for q in (0, 1, 2):
            dl = left_rdma[q]

            @pl.when(q < left_n)
            def _():
                dl.wait_send()

        for q in (1, 2, 3):
            do = own_rdma[q]

            @pl.when(q >= left_n)
            def _():
                do.wait_send()

        for (d, q), fw in fwd_rdma.items():

            @pl.when((q >= left_n) & (d < left_n))
            def _():
                fw.wait_send()

    return pl.pallas_call(
        body,
        out_shape=jax.ShapeDtypeStruct(x.shape, jnp.bfloat16),
        in_specs=[
            pl.BlockSpec(memory_space=pl.ANY),
            pl.BlockSpec(memory_space=pltpu.SMEM),
        ],
        out_specs=pl.BlockSpec(memory_space=pltpu.VMEM),
        scratch_shapes=[
            pltpu.VMEM((2, urows, n), x.dtype),
            pltpu.VMEM((U, prow, n), jnp.int8),
            pltpu.VMEM((3, prow, n), jnp.int8),
            pltpu.VMEM((6, prow, n), jnp.int8),
            pltpu.SemaphoreType.DMA((2,)),
            pltpu.SemaphoreType.DMA((3,)),
            pltpu.SemaphoreType.DMA((3,)),
            pltpu.SemaphoreType.DMA((3, 3)),
            pltpu.SemaphoreType.DMA((3, 3)),
        ],
        compiler_params=pltpu.CompilerParams(
            vmem_limit_bytes=100 * 1024 * 1024,
        ),
    )(x, pi)


# device time: 101062 ns/iter; 1.3451x vs baseline; 1.0582x over previous
import jax
import jax.numpy as jnp
from jax import lax
from jax.experimental import pallas as pl
from jax.experimental.pallas import tpu as pltpu

N_DEV = 4
K = 8

RIDX = {}
for _k in range(2, K):
    for _a in range(_k // 2):
        RIDX[(_a, _k)] = len(RIDX)

QUANT_ORDER = (0, 6, 1, 7, 2, 3, 4, 5)

REV_GROUP = (6, 7, 2, 3, 4, 5)
CONSUME = (
    [("rev", 1, k) for k in REV_GROUP]
    + [("left", 0), ("left", 1)]
    + [("rev", 2, k) for k in REV_GROUP]
    + [("left", 2), ("left", 3)]
    + [("rev", 3, k) for k in (6, 7)]
    + [("left", 4), ("left", 5)]
)


def kernel(x, pi):
    _, m, n = x.shape
    crows = m // K
    prow = crows + 4

    def body(x_ref, pi_ref, out_ref, load_buf, ubuf, lrcv, rbuf,
             load_sems, lsend_sems, lrecv_sems, fsend_sems, frecv_sems):
        my = lax.axis_index("i")
        target = pi_ref[my]
        r = lax.rem(my - target + N_DEV, N_DEV)
        left_n = N_DEV - r
        nleft = 2 * left_n
        rn = lax.rem(my + 1, N_DEV)

        def start_load(k, slot):
            cp = pltpu.make_async_copy(
                x_ref.at[0, pl.ds(k * crows, crows), :],
                load_buf.at[slot],
                load_sems.at[slot],
            )
            cp.start()
            return cp

        def dequant(buf, k):
            data = buf[pl.ds(0, crows), :]
            sc = pltpu.bitcast(buf[pl.ds(crows, 4), :], jnp.float32)
            out_ref[0, pl.ds(k * crows, crows), :] = (
                data.astype(jnp.bfloat16) * sc.astype(jnp.bfloat16)
            )

        loads = {0: start_load(QUANT_ORDER[0], 0), 1: start_load(QUANT_ORDER[1], 1)}
        left_rdma = {}
        own_rdma = {}
        for i in range(K):
            k = QUANT_ORDER[i]
            slot = i % 2
            loads[i].wait()
            a = load_buf[slot]
            am = jnp.maximum(jnp.max(jnp.abs(a), axis=0, keepdims=True), 1e-30)
            ubuf[k, pl.ds(0, crows), :] = (
                jnp.round(a * (127.0 / am)).astype(jnp.int8)
            )
            ubuf[k, pl.ds(crows, 4), :] = pltpu.bitcast(
                am * (1.0 / 127.0), jnp.int8
            )
            if i + 2 < K:
                loads[i + 2] = start_load(QUANT_ORDER[i + 2], slot)
            if k <= 5:
                dl = pltpu.make_async_remote_copy(
                    src_ref=ubuf.at[k],
                    dst_ref=lrcv.at[k],
                    send_sem=lsend_sems.at[k],
                    recv_sem=lrecv_sems.at[k],
                    device_id=(target,),
                    device_id_type=pl.DeviceIdType.MESH,
                )
                left_rdma[k] = dl

                @pl.when(k < nleft)
                def _():
                    dl.start()

            if k >= 2:
                do = pltpu.make_async_remote_copy(
                    src_ref=ubuf.at[k],
                    dst_ref=rbuf.at[RIDX[(0, k)]],
                    send_sem=fsend_sems.at[RIDX[(0, k)]],
                    recv_sem=frecv_sems.at[RIDX[(0, k)]],
                    device_id=(rn,),
                    device_id_type=pl.DeviceIdType.MESH,
                )
                own_rdma[k] = do

                @pl.when(k >= nleft)
                def _():
                    do.start()

        fwd_rdma = {}
        for ev in CONSUME:
            if ev[0] == "left":
                j = ev[1]
                dl = left_rdma[j]

                @pl.when(j < nleft)
                def _():
                    dl.wait_recv()
                    dequant(lrcv.at[j], j)
                continue

            _, d, k = ev
            if d > k // 2:
                continue
            lives = (k >= nleft) & (d <= left_n)
            arr = pltpu.make_async_remote_copy(
                src_ref=rbuf.at[RIDX[(d - 1, k)]],
                dst_ref=rbuf.at[RIDX[(d - 1, k)]],
                send_sem=fsend_sems.at[RIDX[(d - 1, k)]],
                recv_sem=frecv_sems.at[RIDX[(d - 1, k)]],
                device_id=(rn,),
                device_id_type=pl.DeviceIdType.MESH,
            )

            @pl.when(lives)
            def _():
                arr.wait_recv()

            if d <= k // 2 - 1:
                fw = pltpu.make_async_remote_copy(
                    src_ref=rbuf.at[RIDX[(d - 1, k)]],
                    dst_ref=rbuf.at[RIDX[(d, k)]],
                    send_sem=fsend_sems.at[RIDX[(d, k)]],
                    recv_sem=frecv_sems.at[RIDX[(d, k)]],
                    device_id=(rn,),
                    device_id_type=pl.DeviceIdType.MESH,
                )
                fwd_rdma[(d, k)] = fw

                @pl.when((k >= nleft) & (d < left_n))
                def _():
                    fw.start()

            @pl.when((k >= nleft) & (d == left_n))
            def _():
                dequant(rbuf.at[RIDX[(d - 1, k)]], k)

        for k in range(6):
            dl = left_rdma[k]

            @pl.when(k < nleft)
            def _():
                dl.wait_send()

        for k in range(2, K):
            do = own_rdma[k]

            @pl.when(k >= nleft)
            def _():
                do.wait_send()

        for (d, k), fw in fwd_rdma.items():

            @pl.when((k >= nleft) & (d < left_n))
            def _():
                fw.wait_send()

    nrev = len(RIDX)
    return pl.pallas_call(
        body,
        out_shape=jax.ShapeDtypeStruct(x.shape, jnp.bfloat16),
        in_specs=[
            pl.BlockSpec(memory_space=pl.ANY),
            pl.BlockSpec(memory_space=pltpu.SMEM),
        ],
        out_specs=pl.BlockSpec(memory_space=pltpu.VMEM),
        scratch_shapes=[
            pltpu.VMEM((2, crows, n), x.dtype),
            pltpu.VMEM((K, prow, n), jnp.int8),
            pltpu.VMEM((6, prow, n), jnp.int8),
            pltpu.VMEM((nrev, prow, n), jnp.int8),
            pltpu.SemaphoreType.DMA((2,)),
            pltpu.SemaphoreType.DMA((6,)),
            pltpu.SemaphoreType.DMA((6,)),
            pltpu.SemaphoreType.DMA((nrev,)),
            pltpu.SemaphoreType.DMA((nrev,)),
        ],
        compiler_params=pltpu.CompilerParams(
            vmem_limit_bytes=100 * 1024 * 1024,
        ),
    )(x, pi)
